# baseline (speedup 1.0000x reference)
import functools

import jax
import jax.numpy as jnp
from jax import lax
from jax.experimental import pallas as pl
from jax.experimental.pallas import tpu as pltpu
from jax.experimental.pallas import tpu_sc as plsc

_NE = 1_600_000
_NN = 100_000
_D = 32
_DW = _D // 2
_L = 16
_NW = 32
_NS = 16
_C = 80
_K = 5
_B = _C * _K
_PER_W = _NE // _NW
_N_BLOCKS = _PER_W // _B
_N_SUPER = (_N_BLOCKS - 1) // 2
_ROWS_PER_TILE = _NN // _NS


def _tec_kernel(src_hbm, dst_hbm, off_hbm, table_hbm, out_hbm,
                tbl_sp, sidx, didx, srows_a, srows_b, drows_a, drows_b,
                out_a, out_b, off_v,
                sem_sa, sem_sb, sem_da, sem_db, sem_i, sem_oa, sem_ob):
    c = lax.axis_index("c")
    s = lax.axis_index("s")
    wid = s * 2 + c
    base0 = wid * _PER_W
    cbase0 = wid * (_PER_W // _C)

    rbase = s * _ROWS_PER_TILE
    pltpu.sync_copy(table_hbm.at[pl.ds(rbase, _ROWS_PER_TILE)],
                    tbl_sp.at[pl.ds(rbase, _ROWS_PER_TILE)])
    plsc.subcore_barrier()

    pltpu.sync_copy(off_hbm, off_v)
    off_relu = jnp.maximum(off_v[...], 0.0)

    srows = (srows_a, srows_b)
    drows = (drows_a, drows_b)
    sem_s = (sem_sa, sem_sb)
    sem_d = (sem_da, sem_db)

    iota = lax.iota(jnp.int32, _L)

    def issue_gathers(islot, k, buf):
        pltpu.async_copy(
            tbl_sp.at[sidx.at[islot, k]], srows[buf], sem_s[buf])
        pltpu.async_copy(
            tbl_sp.at[didx.at[islot, k]], drows[buf], sem_d[buf])

    def wait_gathers(buf):
        pltpu.make_async_copy(
            tbl_sp.at[sidx.at[0, 0]], srows[buf], sem_s[buf]).wait()
        pltpu.make_async_copy(
            tbl_sp.at[didx.at[0, 0]], drows[buf], sem_d[buf]).wait()

    def issue_idx(b_next, slot):
        cb = cbase0 + b_next * _K
        pltpu.async_copy(src_hbm.at[pl.ds(cb, _K)], sidx.at[slot], sem_i)
        pltpu.async_copy(dst_hbm.at[pl.ds(cb, _K)], didx.at[slot], sem_i)

    def wait_idx(slot):
        pltpu.make_async_copy(
            src_hbm.at[pl.ds(0, _K)], sidx.at[slot], sem_i).wait()
        pltpu.make_async_copy(
            dst_hbm.at[pl.ds(0, _K)], didx.at[slot], sem_i).wait()

    def issue_out(b, out_ref, sem):
        pltpu.async_copy(
            out_ref, out_hbm.at[pl.ds(base0 + b * _B, _B)], sem)

    def wait_out(out_ref, sem):
        pltpu.make_async_copy(
            out_ref, out_hbm.at[pl.ds(base0, _B)], sem).wait()

    def compute(k, buf, out_ref):
        sr = srows[buf]
        dr = drows[buf]

        def group_body(g, carry2):
            row = iota + g * _L
            acc0 = jnp.zeros((_L,), jnp.float32)
            acc1 = jnp.zeros((_L,), jnp.float32)
            acc2 = jnp.zeros((_L,), jnp.float32)
            acc3 = jnp.zeros((_L,), jnp.float32)
            accs = [acc0, acc1, acc2, acc3]
            for w in range(_DW):
                col = jnp.bitwise_and(iota + w, _DW - 1)
                sv = plsc.load_gather(sr, [row, col])
                dv = plsc.load_gather(dr, [row, col])
                sbf = plsc.bitcast(sv, jnp.bfloat16)
                dbf = plsc.bitcast(dv, jnp.bfloat16)
                ubf = sbf - dbf
                ui = plsc.bitcast(ubf, jnp.int32)
                ue = lax.bitcast_convert_type(
                    lax.shift_left(ui, 16), jnp.float32)
                uo = lax.bitcast_convert_type(
                    jnp.bitwise_and(ui, jnp.int32(-65536)), jnp.float32)
                accs[(2 * w) % 4] = accs[(2 * w) % 4] + ue * ue
                accs[(2 * w + 1) % 4] = accs[(2 * w + 1) % 4] + uo * uo
            x = (accs[0] + accs[1]) + (accs[2] + accs[3])
            xc = jnp.maximum(x, 1e-30)
            ib = lax.bitcast_convert_type(xc, jnp.int32)
            ib = 0x5F3759DF - lax.shift_right_logical(ib, 1)
            r = lax.bitcast_convert_type(ib, jnp.float32)
            hx = 0.5 * xc
            r = r * (1.5 - hx * r * r)
            r = r * (1.5 - hx * r * r)
            r = r * (1.5 - hx * r * r)
            dist = xc * r
            out_ref[pl.ds(k * _C + g * _L, _L)] = jnp.exp(off_relu - dist)
            return carry2

        lax.fori_loop(0, _C // _L, group_body, 0)

    def run_block(b, islot, parity0, out_ref, out_sem, next_islot, last):
        for k in range(_K):
            buf = (parity0 + k) % 2
            if k + 1 < _K:
                issue_gathers(islot, k + 1, (buf + 1) % 2)
            elif not last:
                wait_idx(next_islot)
                issue_gathers(next_islot, 0, (buf + 1) % 2)
            wait_gathers(buf)
            compute(k, buf, out_ref)
        issue_out(b, out_ref, out_sem)

    pltpu.sync_copy(src_hbm.at[pl.ds(cbase0, _K)], sidx.at[0])
    pltpu.sync_copy(dst_hbm.at[pl.ds(cbase0, _K)], didx.at[0])
    issue_gathers(0, 0, 0)

    def super_body(u, carry):
        bA = 2 * u
        bB = bA + 1

        @pl.when(u > 0)
        def _():
            wait_out(out_a, sem_oa)
        issue_idx(bA + 1, 1)
        run_block(bA, 0, 0, out_a, sem_oa, 1, last=False)

        @pl.when(u > 0)
        def _():
            wait_out(out_b, sem_ob)
        issue_idx(bB + 1, 0)
        run_block(bB, 1, 1, out_b, sem_ob, 0, last=False)
        return carry

    lax.fori_loop(0, _N_SUPER, super_body, 0)

    wait_out(out_a, sem_oa)
    run_block(_N_BLOCKS - 1, 0, 0, out_a, sem_oa, 1, last=True)
    wait_out(out_a, sem_oa)
    wait_out(out_b, sem_ob)


_mesh = plsc.VectorSubcoreMesh(core_axis_name="c", subcore_axis_name="s")

_poisson_sc = functools.partial(
    pl.kernel,
    mesh=_mesh,
    compiler_params=pltpu.CompilerParams(
        needs_layout_passes=False, use_tc_tiling_on_sc=False),
    out_type=jax.ShapeDtypeStruct((_NE,), jnp.float32),
    scratch_types=[
        pltpu.VMEM_SHARED((_NN, _DW), jnp.int32),
        pltpu.VMEM((2, _K, _C), jnp.int32),
        pltpu.VMEM((2, _K, _C), jnp.int32),
        pltpu.VMEM((_C, _DW), jnp.int32),
        pltpu.VMEM((_C, _DW), jnp.int32),
        pltpu.VMEM((_C, _DW), jnp.int32),
        pltpu.VMEM((_C, _DW), jnp.int32),
        pltpu.VMEM((_B,), jnp.float32),
        pltpu.VMEM((_B,), jnp.float32),
        pltpu.VMEM((_L,), jnp.float32),
        pltpu.SemaphoreType.DMA,
        pltpu.SemaphoreType.DMA,
        pltpu.SemaphoreType.DMA,
        pltpu.SemaphoreType.DMA,
        pltpu.SemaphoreType.DMA,
        pltpu.SemaphoreType.DMA,
        pltpu.SemaphoreType.DMA,
    ],
)(_tec_kernel)


def kernel(src, dst, offset, embedding):
    src2d = src.astype(jnp.int32).reshape(-1, _C)
    dst2d = dst.astype(jnp.int32).reshape(-1, _C)
    ti = lax.bitcast_convert_type(embedding, jnp.int32)
    even = ti[:, 0::2]
    odd = ti[:, 1::2]
    half = jnp.int32(0x8000)
    lo = lax.shift_right_logical(even + half, 16)
    hi = jnp.bitwise_and(odd + half, jnp.int32(-65536))
    table_packed = jnp.bitwise_or(hi, lo)
    off16 = jnp.broadcast_to(offset.astype(jnp.float32), (_L,))
    return _poisson_sc(src2d, dst2d, off16, table_packed)


# speedup vs baseline: 2.8730x; 2.8730x over previous
import functools

import jax
import jax.numpy as jnp
from jax import lax
from jax.experimental import pallas as pl
from jax.experimental.pallas import tpu as pltpu
from jax.experimental.pallas import tpu_sc as plsc

_NE = 1_600_000
_NN = 100_000
_D = 32
_DW = _D // 2
_L = 16
_NW = 32
_NS = 16
_C = 80
_K = 5
_B = _C * _K
_PER_W = _NE // _NW
_N_BLOCKS = _PER_W // _B
_N_SUPER = (_N_BLOCKS - 1) // 2
_ROWS_PER_TILE = _NN // _NS
_CVT_ROWS = 250


def _tec_kernel(src_hbm, dst_hbm, off_hbm, table_hbm, out_hbm,
                tbl_sp, sidx, didx, srows_a, srows_b, drows_a, drows_b,
                out_a, out_b, off_v, cvt_in, cvt_out,
                sem_sa, sem_sb, sem_da, sem_db, sem_i, sem_oa, sem_ob):
    c = lax.axis_index("c")
    s = lax.axis_index("s")
    wid = s * 2 + c
    base0 = wid * _PER_W
    cbase0 = wid * (_PER_W // _C)

    rbase = s * _ROWS_PER_TILE

    def cvt_block(t, carry):
        rb = rbase + t * _CVT_ROWS
        pltpu.sync_copy(table_hbm.at[pl.ds(rb, _CVT_ROWS)], cvt_in)

        def row_body(r, carry2):
            v0 = cvt_in[r, pl.ds(0, _L)]
            v1 = cvt_in[r, pl.ds(_L, _L)]
            packed = plsc.bitcast(
                plsc.pack(v0, v1, format=plsc.PackFormat.INTERLEAVED),
                jnp.int32)
            cvt_out[r, :] = packed
            return carry2

        lax.fori_loop(0, _CVT_ROWS, row_body, 0, unroll=4)
        pltpu.sync_copy(cvt_out, tbl_sp.at[pl.ds(rb, _CVT_ROWS)])
        return carry

    lax.fori_loop(0, _ROWS_PER_TILE // _CVT_ROWS, cvt_block, 0)
    plsc.subcore_barrier()

    pltpu.sync_copy(off_hbm, off_v)
    off_relu = jnp.maximum(off_v[...], 0.0)

    srows = (srows_a, srows_b)
    drows = (drows_a, drows_b)
    sem_s = (sem_sa, sem_sb)
    sem_d = (sem_da, sem_db)

    iota = lax.iota(jnp.int32, _L)

    def issue_gathers(islot, k, buf):
        pltpu.async_copy(
            tbl_sp.at[sidx.at[islot, k]], srows[buf], sem_s[buf])
        pltpu.async_copy(
            tbl_sp.at[didx.at[islot, k]], drows[buf], sem_d[buf])

    def wait_gathers(buf):
        pltpu.make_async_copy(
            tbl_sp.at[sidx.at[0, 0]], srows[buf], sem_s[buf]).wait()
        pltpu.make_async_copy(
            tbl_sp.at[didx.at[0, 0]], drows[buf], sem_d[buf]).wait()

    def issue_idx(b_next, slot):
        cb = cbase0 + b_next * _K
        pltpu.async_copy(src_hbm.at[pl.ds(cb, _K)], sidx.at[slot], sem_i)
        pltpu.async_copy(dst_hbm.at[pl.ds(cb, _K)], didx.at[slot], sem_i)

    def wait_idx(slot):
        pltpu.make_async_copy(
            src_hbm.at[pl.ds(0, _K)], sidx.at[slot], sem_i).wait()
        pltpu.make_async_copy(
            dst_hbm.at[pl.ds(0, _K)], didx.at[slot], sem_i).wait()

    def issue_out(b, out_ref, sem):
        pltpu.async_copy(
            out_ref, out_hbm.at[pl.ds(base0 + b * _B, _B)], sem)

    def wait_out(out_ref, sem):
        pltpu.make_async_copy(
            out_ref, out_hbm.at[pl.ds(base0, _B)], sem).wait()

    def compute(k, buf, out_ref):
        sr = srows[buf]
        dr = drows[buf]

        def group_body(g, carry2):
            row = iota + g * _L
            acc0 = jnp.zeros((_L,), jnp.float32)
            acc1 = jnp.zeros((_L,), jnp.float32)
            acc2 = jnp.zeros((_L,), jnp.float32)
            acc3 = jnp.zeros((_L,), jnp.float32)
            accs = [acc0, acc1, acc2, acc3]
            for w in range(_DW):
                col = jnp.bitwise_and(iota + w, _DW - 1)
                sv = plsc.load_gather(sr, [row, col])
                dv = plsc.load_gather(dr, [row, col])
                sbf = plsc.bitcast(sv, jnp.bfloat16)
                dbf = plsc.bitcast(dv, jnp.bfloat16)
                ubf = sbf - dbf
                ui = plsc.bitcast(ubf, jnp.int32)
                ue = lax.bitcast_convert_type(
                    lax.shift_left(ui, 16), jnp.float32)
                uo = lax.bitcast_convert_type(
                    jnp.bitwise_and(ui, jnp.int32(-65536)), jnp.float32)
                accs[(2 * w) % 4] = accs[(2 * w) % 4] + ue * ue
                accs[(2 * w + 1) % 4] = accs[(2 * w + 1) % 4] + uo * uo
            x = (accs[0] + accs[1]) + (accs[2] + accs[3])
            xc = jnp.maximum(x, 1e-30)
            ib = lax.bitcast_convert_type(xc, jnp.int32)
            ib = 0x5F3759DF - lax.shift_right_logical(ib, 1)
            r = lax.bitcast_convert_type(ib, jnp.float32)
            hx = 0.5 * xc
            r = r * (1.5 - hx * r * r)
            r = r * (1.5 - hx * r * r)
            r = r * (1.5 - hx * r * r)
            dist = xc * r
            out_ref[pl.ds(k * _C + g * _L, _L)] = jnp.exp(off_relu - dist)
            return carry2

        lax.fori_loop(0, _C // _L, group_body, 0)

    def run_block(b, islot, parity0, out_ref, out_sem, next_islot, last):
        for k in range(_K):
            buf = (parity0 + k) % 2
            if k + 1 < _K:
                issue_gathers(islot, k + 1, (buf + 1) % 2)
            elif not last:
                wait_idx(next_islot)
                issue_gathers(next_islot, 0, (buf + 1) % 2)
            wait_gathers(buf)
            compute(k, buf, out_ref)
        issue_out(b, out_ref, out_sem)

    pltpu.sync_copy(src_hbm.at[pl.ds(cbase0, _K)], sidx.at[0])
    pltpu.sync_copy(dst_hbm.at[pl.ds(cbase0, _K)], didx.at[0])
    issue_gathers(0, 0, 0)

    def super_body(u, carry):
        bA = 2 * u
        bB = bA + 1

        @pl.when(u > 0)
        def _():
            wait_out(out_a, sem_oa)
        issue_idx(bA + 1, 1)
        run_block(bA, 0, 0, out_a, sem_oa, 1, last=False)

        @pl.when(u > 0)
        def _():
            wait_out(out_b, sem_ob)
        issue_idx(bB + 1, 0)
        run_block(bB, 1, 1, out_b, sem_ob, 0, last=False)
        return carry

    lax.fori_loop(0, _N_SUPER, super_body, 0)

    wait_out(out_a, sem_oa)
    run_block(_N_BLOCKS - 1, 0, 0, out_a, sem_oa, 1, last=True)
    wait_out(out_a, sem_oa)
    wait_out(out_b, sem_ob)


_mesh = plsc.VectorSubcoreMesh(core_axis_name="c", subcore_axis_name="s")

_poisson_sc = functools.partial(
    pl.kernel,
    mesh=_mesh,
    compiler_params=pltpu.CompilerParams(
        needs_layout_passes=False, use_tc_tiling_on_sc=False),
    out_type=jax.ShapeDtypeStruct((_NE,), jnp.float32),
    scratch_types=[
        pltpu.VMEM_SHARED((_NN, _DW), jnp.int32),
        pltpu.VMEM((2, _K, _C), jnp.int32),
        pltpu.VMEM((2, _K, _C), jnp.int32),
        pltpu.VMEM((_C, _DW), jnp.int32),
        pltpu.VMEM((_C, _DW), jnp.int32),
        pltpu.VMEM((_C, _DW), jnp.int32),
        pltpu.VMEM((_C, _DW), jnp.int32),
        pltpu.VMEM((_B,), jnp.float32),
        pltpu.VMEM((_B,), jnp.float32),
        pltpu.VMEM((_L,), jnp.float32),
        pltpu.VMEM((_CVT_ROWS, _D), jnp.float32),
        pltpu.VMEM((_CVT_ROWS, _DW), jnp.int32),
        pltpu.SemaphoreType.DMA,
        pltpu.SemaphoreType.DMA,
        pltpu.SemaphoreType.DMA,
        pltpu.SemaphoreType.DMA,
        pltpu.SemaphoreType.DMA,
        pltpu.SemaphoreType.DMA,
        pltpu.SemaphoreType.DMA,
    ],
)(_tec_kernel)


def kernel(src, dst, offset, embedding):
    src2d = src.astype(jnp.int32).reshape(-1, _C)
    dst2d = dst.astype(jnp.int32).reshape(-1, _C)
    off16 = jnp.broadcast_to(offset.astype(jnp.float32), (_L,))
    return _poisson_sc(src2d, dst2d, off16, embedding)
